# baseline (speedup 1.0000x reference)
import functools

import jax
import jax.numpy as jnp
from jax import lax
from jax.experimental import pallas as pl
from jax.experimental.pallas import tpu as pltpu
from jax.experimental.pallas import tpu_sc as plsc

B, C, H, W = 4096, 64, 4, 4
D = C * H * W
MAX_ID = 60000
G = MAX_ID // 16
NC, NS, L = 2, 16, 16
NW = NC * NS
BPW = B // NW
CH = 32
NCHUNK = BPW // CH

_mesh = plsc.VectorSubcoreMesh(core_axis_name="c", subcore_axis_name="s")


@functools.partial(
    pl.kernel,
    mesh=_mesh,
    compiler_params=pltpu.CompilerParams(needs_layout_passes=False),
    out_type=jax.ShapeDtypeStruct((B, D), jnp.float32),
    scratch_types=[
        pltpu.VMEM((BPW,), jnp.int32),
        pltpu.VMEM((BPW,), jnp.int32),
        pltpu.VMEM((CH, D), jnp.float32),
        pltpu.VMEM((CH, D), jnp.float32),
        pltpu.SemaphoreType.DMA,
        pltpu.SemaphoreType.DMA,
    ],
)
def _tied_dropout(x_hbm, idx_hbm, packed_hbm, out_hbm,
                  idx_v, g_v, p_v, x_v, psem, xsem):
    wid = lax.axis_index("s") * NC + lax.axis_index("c")
    base = wid * BPW
    pltpu.sync_copy(idx_hbm.at[pl.ds(base, BPW)], idx_v)

    def gv_body(j, _):
        g_v[pl.ds(j * L, L)] = idx_v[pl.ds(j * L, L)] >> 4
        return 0

    lax.fori_loop(0, BPW // L, gv_body, 0)

    for k in range(NCHUNK):
        row0 = base + k * CH
        pc = pltpu.async_copy(
            packed_hbm.at[g_v.at[pl.ds(k * CH, CH)]], p_v, psem)
        xc = pltpu.async_copy(x_hbm.at[pl.ds(row0, CH)], x_v, xsem)
        pc.wait()
        xc.wait()

        def row_body(r, _):
            jsplat = plsc.load_gather(
                idx_v, [jnp.full((L,), k * CH, jnp.int32) + r])
            jv = jsplat & 15

            def col_body(c, _):
                c0 = c * L
                pw = p_v[r, pl.ds(c0, L)].astype(jnp.int32)
                b = ((pw >> jv) & 1).astype(jnp.float32)
                x_v[r, pl.ds(c0, L)] = x_v[r, pl.ds(c0, L)] * b
                return 0

            lax.fori_loop(0, D // L, col_body, 0)
            return 0

        lax.fori_loop(0, CH, row_body, 0)
        pltpu.sync_copy(x_v, out_hbm.at[pl.ds(row0, CH)])


def kernel(X, idx, mask_tensor):
    t = mask_tensor
    s = t[0::2] + 2.0 * t[1::2]
    s = s[0::2] + 4.0 * s[1::2]
    s = s[0::2] + 16.0 * s[1::2]
    s = s[0::2] + 256.0 * s[1::2]
    x2 = X.reshape(B, D)
    out = _tied_dropout(x2, idx, s.reshape(G, D))
    return out.reshape(B, C, H, W)


# speedup vs baseline: 23.5965x; 23.5965x over previous
import functools

import jax
import jax.numpy as jnp
from jax import lax
from jax.experimental import pallas as pl
from jax.experimental.pallas import tpu as pltpu
from jax.experimental.pallas import tpu_sc as plsc

B, C, H, W = 4096, 64, 4, 4
D = C * H * W
MAX_ID = 60000
G = MAX_ID // 16
NC, NS, L = 2, 16, 16
NW = NC * NS
BPW = B // NW
CH = 32
NCHUNK = BPW // CH

_mesh = plsc.VectorSubcoreMesh(core_axis_name="c", subcore_axis_name="s")


@functools.partial(
    pl.kernel,
    mesh=_mesh,
    compiler_params=pltpu.CompilerParams(needs_layout_passes=False),
    out_type=jax.ShapeDtypeStruct((B, D), jnp.float32),
    scratch_types=[
        pltpu.VMEM((BPW,), jnp.int32),
        pltpu.VMEM((BPW,), jnp.int32),
        pltpu.VMEM((CH, D), jnp.float32),
        pltpu.VMEM((CH, D), jnp.float32),
        pltpu.SemaphoreType.DMA,
        pltpu.SemaphoreType.DMA,
    ],
)
def _tied_dropout(x_hbm, idx_hbm, packed_hbm, out_hbm,
                  idx_v, g_v, p_v, x_v, psem, xsem):
    wid = lax.axis_index("s") * NC + lax.axis_index("c")
    base = wid * BPW
    pltpu.sync_copy(idx_hbm.at[pl.ds(base, BPW)], idx_v)

    def gv_body(j, _):
        g_v[pl.ds(j * L, L)] = idx_v[pl.ds(j * L, L)] >> 4
        return 0

    lax.fori_loop(0, BPW // L, gv_body, 0)

    for k in range(NCHUNK):
        row0 = base + k * CH
        pc = pltpu.async_copy(
            packed_hbm.at[g_v.at[pl.ds(k * CH, CH)]], p_v, psem)
        xc = pltpu.async_copy(x_hbm.at[pl.ds(row0, CH)], x_v, xsem)
        pc.wait()
        xc.wait()

        def row_body(r, _):
            jsplat = plsc.load_gather(
                idx_v, [jnp.full((L,), k * CH, jnp.int32) + r])
            jv = jsplat & 15

            def col_body(c, _):
                c0 = c * L
                pw = p_v[r, pl.ds(c0, L)].astype(jnp.int32)
                b = ((pw >> jv) & 1).astype(jnp.float32)
                x_v[r, pl.ds(c0, L)] = x_v[r, pl.ds(c0, L)] * b
                return 0

            lax.fori_loop(0, D // L, col_body, 0)
            return 0

        lax.fori_loop(0, CH, row_body, 0)
        pltpu.sync_copy(x_v, out_hbm.at[pl.ds(row0, CH)])


def kernel(X, idx, mask_tensor):
    pat = jnp.tile(jnp.exp2(jnp.arange(16, dtype=jnp.float32)), MAX_ID // 16)
    weighted = mask_tensor * pat[:, None, None, None]
    s = lax.reduce_window(weighted, 0.0, lax.add,
                          window_dimensions=(16, 1, 1, 1),
                          window_strides=(16, 1, 1, 1),
                          padding="VALID")
    x2 = X.reshape(B, D)
    out = _tied_dropout(x2, idx, s.reshape(G, D))
    return out.reshape(B, C, H, W)


# speedup vs baseline: 28.4829x; 1.2071x over previous
import functools

import jax
import jax.numpy as jnp
from jax import lax
from jax.experimental import pallas as pl
from jax.experimental.pallas import tpu as pltpu
from jax.experimental.pallas import tpu_sc as plsc

B, C, H, W = 4096, 64, 4, 4
D = C * H * W
COFF = 128
DM = D - COFF
MAX_ID = 60000
NC, NS, L = 2, 16, 16
NW = NC * NS
BPW = B // NW
CH = 16
NCHUNK = BPW // CH

_mesh = plsc.VectorSubcoreMesh(core_axis_name="c", subcore_axis_name="s")


@functools.partial(
    pl.kernel,
    mesh=_mesh,
    out_type=jax.ShapeDtypeStruct((B, D), jnp.float32),
    scratch_types=[
        pltpu.VMEM((BPW,), jnp.int32),
        pltpu.VMEM((CH, DM), jnp.float32),
        pltpu.VMEM((CH, DM), jnp.float32),
        pltpu.VMEM((CH, D), jnp.float32),
        pltpu.VMEM((CH, D), jnp.float32),
        pltpu.SemaphoreType.DMA,
        pltpu.SemaphoreType.DMA,
        pltpu.SemaphoreType.DMA,
        pltpu.SemaphoreType.DMA,
        pltpu.SemaphoreType.DMA,
        pltpu.SemaphoreType.DMA,
    ],
)
def _tied_dropout(x_hbm, idx_hbm, table_hbm, out_hbm,
                  idx_v, m0, m1, x0, x1,
                  gs0, gs1, xs0, xs1, os0, os1):
    mbuf = (m0, m1)
    xbuf = (x0, x1)
    gsem = (gs0, gs1)
    xsem = (xs0, xs1)
    osem = (os0, os1)
    wid = lax.axis_index("s") * NC + lax.axis_index("c")
    base = wid * BPW
    pltpu.sync_copy(idx_hbm.at[pl.ds(base, BPW)], idx_v)

    def start(k):
        b = k % 2
        return (
            pltpu.async_copy(
                table_hbm.at[idx_v.at[pl.ds(k * CH, CH)]], mbuf[b], gsem[b]),
            pltpu.async_copy(
                x_hbm.at[pl.ds(base + k * CH, CH)], xbuf[b], xsem[b]),
        )

    inflight = start(0)
    outflight = [None, None]
    for k in range(NCHUNK):
        b = k % 2
        nb = (k + 1) % 2
        if k + 1 < NCHUNK:
            if outflight[nb] is not None:
                outflight[nb].wait()
                outflight[nb] = None
            nxt = start(k + 1)
        gc, xc = inflight
        gc.wait()
        xc.wait()

        def row_body(r, _):
            def col_body(c, _):
                c0 = c * L
                xslc = x_v_cur[r, pl.ds(COFF + c0, L)]
                x_v_cur[r, pl.ds(COFF + c0, L)] = xslc * m_v_cur[r, pl.ds(c0, L)]
                return 0

            lax.fori_loop(0, DM // L, col_body, 0)
            return 0

        m_v_cur = mbuf[b]
        x_v_cur = xbuf[b]
        lax.fori_loop(0, CH, row_body, 0)
        outflight[b] = pltpu.async_copy(
            xbuf[b], out_hbm.at[pl.ds(base + k * CH, CH)], osem[b])
        if k + 1 < NCHUNK:
            inflight = nxt
    for b in range(2):
        if outflight[b] is not None:
            outflight[b].wait()


def kernel(X, idx, mask_tensor):
    table = mask_tensor.reshape(MAX_ID, D)[:, COFF:]
    x2 = X.reshape(B, D)
    out = _tied_dropout(x2, idx, table)
    return out.reshape(B, C, H, W)


# speedup vs baseline: 34.5082x; 1.2115x over previous
import functools

import jax
import jax.numpy as jnp
from jax import lax
from jax.experimental import pallas as pl
from jax.experimental.pallas import tpu as pltpu
from jax.experimental.pallas import tpu_sc as plsc

B, C, H, W = 4096, 64, 4, 4
D = C * H * W
COFF = 0
DM = D - COFF
MAX_ID = 60000
NC, NS, L = 2, 16, 16
NW = NC * NS
BPW = B // NW
CH = 16
NCHUNK = BPW // CH

_mesh = plsc.VectorSubcoreMesh(core_axis_name="c", subcore_axis_name="s")


@functools.partial(
    pl.kernel,
    mesh=_mesh,
    out_type=jax.ShapeDtypeStruct((B, D), jnp.float32),
    scratch_types=[
        pltpu.VMEM((BPW,), jnp.int32),
        pltpu.VMEM((CH, DM), jnp.float32),
        pltpu.VMEM((CH, DM), jnp.float32),
        pltpu.VMEM((CH, D), jnp.float32),
        pltpu.VMEM((CH, D), jnp.float32),
        pltpu.SemaphoreType.DMA,
        pltpu.SemaphoreType.DMA,
        pltpu.SemaphoreType.DMA,
        pltpu.SemaphoreType.DMA,
        pltpu.SemaphoreType.DMA,
        pltpu.SemaphoreType.DMA,
    ],
)
def _tied_dropout(x_hbm, idx_hbm, table_hbm, out_hbm,
                  idx_v, m0, m1, x0, x1,
                  gs0, gs1, xs0, xs1, os0, os1):
    mbuf = (m0, m1)
    xbuf = (x0, x1)
    gsem = (gs0, gs1)
    xsem = (xs0, xs1)
    osem = (os0, os1)
    wid = lax.axis_index("s") * NC + lax.axis_index("c")
    base = wid * BPW
    pltpu.sync_copy(idx_hbm.at[pl.ds(base, BPW)], idx_v)

    def start(k):
        b = k % 2
        return (
            pltpu.async_copy(
                table_hbm.at[idx_v.at[pl.ds(k * CH, CH)]], mbuf[b], gsem[b]),
            pltpu.async_copy(
                x_hbm.at[pl.ds(base + k * CH, CH)], xbuf[b], xsem[b]),
        )

    inflight = start(0)
    outflight = [None, None]
    for k in range(NCHUNK):
        b = k % 2
        nb = (k + 1) % 2
        if k + 1 < NCHUNK:
            if outflight[nb] is not None:
                outflight[nb].wait()
                outflight[nb] = None
            nxt = start(k + 1)
        gc, xc = inflight
        gc.wait()
        xc.wait()

        def row_body(r, _):
            def col_body(c, _):
                c0 = c * L
                xslc = x_v_cur[r, pl.ds(COFF + c0, L)]
                x_v_cur[r, pl.ds(COFF + c0, L)] = xslc * m_v_cur[r, pl.ds(c0, L)]
                return 0

            lax.fori_loop(0, DM // L, col_body, 0)
            return 0

        m_v_cur = mbuf[b]
        x_v_cur = xbuf[b]
        lax.fori_loop(0, CH, row_body, 0)
        outflight[b] = pltpu.async_copy(
            xbuf[b], out_hbm.at[pl.ds(base + k * CH, CH)], osem[b])
        if k + 1 < NCHUNK:
            inflight = nxt
    for b in range(2):
        if outflight[b] is not None:
            outflight[b].wait()


def kernel(X, idx, mask_tensor):
    table = mask_tensor.reshape(MAX_ID, D)
    x2 = X.reshape(B, D)
    out = _tied_dropout(x2, idx, table)
    return out.reshape(B, C, H, W)
